# Initial kernel scaffold; baseline (speedup 1.0000x reference)
import jax
import jax.numpy as jnp
from jax.experimental import pallas as pl


def kernel(x, t, W_self, W_nbr, w_t):
    raise NotImplementedError("write your pallas kernel here")


# speedup vs baseline: 133.4180x; 133.4180x over previous
import jax
import jax.numpy as jnp
from jax.experimental import pallas as pl
from jax.experimental.pallas import tpu as pltpu

N = 400
_INF_BITS = 0x7F800000


def _gnn_kernel(t_ref, x_ref, ws_ref, wn_ref, wt_ref, out_ref):
    b = pl.program_id(0)
    xb = x_ref[0]

    bits = jax.lax.bitcast_convert_type(xb, jnp.int32)
    posm = xb > 0.0
    n = jnp.sum(posm.astype(jnp.int32))
    sent = jnp.int32(_INF_BITS)
    bi = jnp.where(posm, bits, sent)

    nf = n.astype(jnp.float32)
    pos = jnp.float32(0.95) * (nf - jnp.float32(1.0))
    lowf = jnp.floor(pos)
    hw = pos - lowf
    lw = jnp.float32(1.0) - hw
    k = lowf.astype(jnp.int32)
    target = k + 1

    def body(_, lohi):
        lo, hi = lohi
        mid = lo + ((hi - lo) >> 1)
        cnt = jnp.sum((bi <= mid).astype(jnp.int32))
        pred = cnt >= target
        return (jnp.where(pred, lo, mid + 1), jnp.where(pred, mid, hi))

    lo, _ = jax.lax.fori_loop(
        0, 31, body, (jnp.int32(0), jnp.int32(_INF_BITS - 1))
    )
    vk = jax.lax.bitcast_convert_type(lo, jnp.float32)
    cnt_k = jnp.sum((bi <= lo).astype(jnp.int32))
    nxt = jnp.min(jnp.where(bi > lo, bi, sent))
    vnext = jax.lax.bitcast_convert_type(nxt, jnp.float32)
    vkp1 = jnp.where(cnt_k >= target + 1, vk, vnext)
    vhi = jnp.where(hw > 0.0, vkp1, vk)
    thr = vk * lw + vhi * hw
    thr = jnp.where(n > 0, thr, jnp.float32(jnp.inf))

    mask = (xb >= thr).astype(jnp.float32)
    agg = jax.lax.dot_general(
        mask, xb, (((0,), (0,)), ((), ())),
        preferred_element_type=jnp.float32,
        precision=jax.lax.Precision.HIGHEST,
    )
    out = jnp.dot(xb, ws_ref[...], preferred_element_type=jnp.float32)
    out = out + jnp.dot(agg, wn_ref[...], preferred_element_type=jnp.float32)
    out = out + t_ref[b] * wt_ref[0]
    out_ref[0] = out


def kernel(x, t, W_self, W_nbr, w_t):
    B = x.shape[0]
    grid_spec = pltpu.PrefetchScalarGridSpec(
        num_scalar_prefetch=1,
        grid=(B,),
        in_specs=[
            pl.BlockSpec((1, N, N), lambda b, t_s: (b, 0, 0)),
            pl.BlockSpec((N, N), lambda b, t_s: (0, 0)),
            pl.BlockSpec((N, N), lambda b, t_s: (0, 0)),
            pl.BlockSpec((1, N), lambda b, t_s: (0, 0)),
        ],
        out_specs=pl.BlockSpec((1, N, N), lambda b, t_s: (b, 0, 0)),
    )
    out = pl.pallas_call(
        _gnn_kernel,
        grid_spec=grid_spec,
        out_shape=jax.ShapeDtypeStruct((B, N, N), jnp.float32),
    )(t, x, W_self, W_nbr, w_t.reshape(1, N))
    return out


# speedup vs baseline: 329.1533x; 2.4671x over previous
import jax
import jax.numpy as jnp
from jax.experimental import pallas as pl
from jax.experimental.pallas import tpu as pltpu

N = 400
G = 8
_INF_BITS = 0x7F800000


def _gnn_kernel(t_ref, x_ref, ws_ref, wn_ref, wt_ref, out_ref):
    pid = pl.program_id(0)
    xs = x_ref[...]
    bits = jax.lax.bitcast_convert_type(xs, jnp.int32)
    posm = xs > 0.0
    sent = jnp.int32(_INF_BITS)
    bi = jnp.where(posm, bits, sent)

    ns = [jnp.sum(posm[g].astype(jnp.int32)) for g in range(G)]

    targets, hws, lws = [], [], []
    for g in range(G):
        nf = ns[g].astype(jnp.float32)
        pos = jnp.float32(0.95) * (nf - jnp.float32(1.0))
        lowf = jnp.floor(pos)
        hw = pos - lowf
        targets.append(lowf.astype(jnp.int32) + 1)
        hws.append(hw)
        lws.append(jnp.float32(1.0) - hw)

    def body(_, carry):
        los, his = carry
        new_lo, new_hi = [], []
        for g in range(G):
            lo, hi = los[g], his[g]
            mid = lo + ((hi - lo) >> 1)
            cnt = jnp.sum((bi[g] <= mid).astype(jnp.int32))
            pred = cnt >= targets[g]
            new_lo.append(jnp.where(pred, lo, mid + 1))
            new_hi.append(jnp.where(pred, mid, hi))
        return (tuple(new_lo), tuple(new_hi))

    init = (
        tuple(jnp.int32(0) for _ in range(G)),
        tuple(jnp.int32(_INF_BITS - 1) for _ in range(G)),
    )
    los, _ = jax.lax.fori_loop(0, 31, body, init)

    cnt_ks = [jnp.sum((bi[g] <= los[g]).astype(jnp.int32)) for g in range(G)]
    nxts = [jnp.min(jnp.where(bi[g] > los[g], bi[g], sent)) for g in range(G)]

    thrs = []
    for g in range(G):
        vk = jax.lax.bitcast_convert_type(los[g], jnp.float32)
        vnext = jax.lax.bitcast_convert_type(nxts[g], jnp.float32)
        vkp1 = jnp.where(cnt_ks[g] >= targets[g] + 1, vk, vnext)
        vhi = jnp.where(hws[g] > 0.0, vkp1, vk)
        thr = vk * lws[g] + vhi * hws[g]
        thrs.append(jnp.where(ns[g] > 0, thr, jnp.float32(jnp.inf)))

    ws = ws_ref[...]
    wn = wn_ref[...]
    wtrow = wt_ref[...]
    for g in range(G):
        xb = xs[g]
        mask = (xb >= thrs[g]).astype(jnp.float32)
        agg = jax.lax.dot_general(
            mask, xb, (((0,), (0,)), ((), ())),
            preferred_element_type=jnp.float32,
        )
        out = jnp.dot(xb, ws, preferred_element_type=jnp.float32)
        out = out + jnp.dot(agg, wn, preferred_element_type=jnp.float32)
        out = out + t_ref[pid * G + g] * wtrow
        out_ref[g] = out


def kernel(x, t, W_self, W_nbr, w_t):
    B = x.shape[0]
    grid_spec = pltpu.PrefetchScalarGridSpec(
        num_scalar_prefetch=1,
        grid=(B // G,),
        in_specs=[
            pl.BlockSpec((G, N, N), lambda i, t_s: (i, 0, 0)),
            pl.BlockSpec((N, N), lambda i, t_s: (0, 0)),
            pl.BlockSpec((N, N), lambda i, t_s: (0, 0)),
            pl.BlockSpec((1, N), lambda i, t_s: (0, 0)),
        ],
        out_specs=pl.BlockSpec((G, N, N), lambda i, t_s: (i, 0, 0)),
    )
    out = pl.pallas_call(
        _gnn_kernel,
        grid_spec=grid_spec,
        out_shape=jax.ShapeDtypeStruct((B, N, N), jnp.float32),
    )(t, x, W_self, W_nbr, w_t.reshape(1, N))
    return out
